# initial kernel scaffold (unmeasured)
import jax
import jax.numpy as jnp
from jax import lax
from jax.experimental import pallas as pl
from jax.experimental.pallas import tpu as pltpu


def kernel(
    x,
):
    def body(*refs):
        pass

    out_shape = jax.ShapeDtypeStruct(..., jnp.float32)
    return pl.pallas_call(body, out_shape=out_shape)(...)


# baseline (device time: 31327 ns/iter reference)
import jax
import jax.numpy as jnp
from jax import lax
from jax.experimental import pallas as pl
from jax.experimental.pallas import tpu as pltpu


def kernel(x):
    _, m, n2 = x.shape
    n = n2 // 2

    def body(x_ref, out_ref, sbuf, rbuf, send_sem, recv_sem):
        my_x = lax.axis_index("x")
        my_y = lax.axis_index("y")
        peer = (my_x, 1 - my_y)

        barrier = pltpu.get_barrier_semaphore()
        pl.semaphore_signal(
            barrier, inc=1, device_id=peer, device_id_type=pl.DeviceIdType.MESH
        )
        pl.semaphore_wait(barrier, 1)

        @pl.when(my_y == 0)
        def _():
            sbuf[...] = x_ref[0, :, n:].astype(jnp.bfloat16)

        @pl.when(my_y == 1)
        def _():
            sbuf[...] = x_ref[0, :, :n].astype(jnp.bfloat16)

        rdma = pltpu.make_async_remote_copy(
            src_ref=sbuf,
            dst_ref=rbuf,
            send_sem=send_sem,
            recv_sem=recv_sem,
            device_id=peer,
            device_id_type=pl.DeviceIdType.MESH,
        )
        rdma.start()
        rdma.wait()

        @pl.when(my_y == 0)
        def _():
            out_ref[...] = x_ref[0, :, :n].astype(jnp.bfloat16) + rbuf[...]

        @pl.when(my_y == 1)
        def _():
            out_ref[...] = x_ref[0, :, n:].astype(jnp.bfloat16) + rbuf[...]

    return pl.pallas_call(
        body,
        out_shape=jax.ShapeDtypeStruct((m, n), jnp.bfloat16),
        in_specs=[pl.BlockSpec(memory_space=pltpu.VMEM)],
        out_specs=pl.BlockSpec(memory_space=pltpu.VMEM),
        scratch_shapes=[
            pltpu.VMEM((m, n), jnp.bfloat16),
            pltpu.VMEM((m, n), jnp.bfloat16),
            pltpu.SemaphoreType.DMA,
            pltpu.SemaphoreType.DMA,
        ],
        compiler_params=pltpu.CompilerParams(collective_id=0),
    )(x)


# device time: 23773 ns/iter; 1.3178x vs baseline; 1.3178x over previous
import jax
import jax.numpy as jnp
from jax import lax
from jax.experimental import pallas as pl
from jax.experimental.pallas import tpu as pltpu

K = 8


def kernel(x):
    _, m, n2 = x.shape
    n = n2 // 2
    half = m // 2
    ch = half // K

    def body(x_ref, out_ref, sybuf, rybuf, ysend, yrecv, xsend, xrecv):
        my_x = lax.axis_index("x")
        my_y = lax.axis_index("y")
        peer_y = (my_x, 1 - my_y)
        peer_x = (1 - my_x, my_y)
        row0 = my_x * half

        barrier = pltpu.get_barrier_semaphore()
        for p in (peer_y, peer_x):
            pl.semaphore_signal(
                barrier, inc=1, device_id=p, device_id_type=pl.DeviceIdType.MESH
            )
        pl.semaphore_wait(barrier, 2)

        def y_rdma(k):
            return pltpu.make_async_remote_copy(
                src_ref=sybuf.at[pl.ds(k * ch, ch), :],
                dst_ref=rybuf.at[pl.ds(k * ch, ch), :],
                send_sem=ysend.at[k],
                recv_sem=yrecv.at[k],
                device_id=peer_y,
                device_id_type=pl.DeviceIdType.MESH,
            )

        def x_rdma(k):
            return pltpu.make_async_remote_copy(
                src_ref=out_ref.at[pl.ds(row0 + k * ch, ch), :],
                dst_ref=out_ref.at[pl.ds(row0 + k * ch, ch), :],
                send_sem=xsend.at[k],
                recv_sem=xrecv.at[k],
                device_id=peer_x,
                device_id_type=pl.DeviceIdType.MESH,
            )

        def stage_and_send(k):
            rs = row0 + k * ch

            @pl.when(my_y == 0)
            def _():
                sybuf[pl.ds(k * ch, ch), :] = x_ref[0, pl.ds(rs, ch), n:].astype(
                    jnp.bfloat16
                )

            @pl.when(my_y == 1)
            def _():
                sybuf[pl.ds(k * ch, ch), :] = x_ref[0, pl.ds(rs, ch), :n].astype(
                    jnp.bfloat16
                )

            y_rdma(k).start()

        for k in range(K):
            stage_and_send(k)

        def reduce_and_forward(k):
            rs = row0 + k * ch
            y_rdma(k).wait_recv()

            @pl.when(my_y == 0)
            def _():
                out_ref[pl.ds(rs, ch), :] = (
                    x_ref[0, pl.ds(rs, ch), :n].astype(jnp.bfloat16)
                    + rybuf[pl.ds(k * ch, ch), :]
                )

            @pl.when(my_y == 1)
            def _():
                out_ref[pl.ds(rs, ch), :] = (
                    x_ref[0, pl.ds(rs, ch), n:].astype(jnp.bfloat16)
                    + rybuf[pl.ds(k * ch, ch), :]
                )

            x_rdma(k).start()

        for k in range(K):
            reduce_and_forward(k)

        for k in range(K):
            y_rdma(k).wait_send()
            x_rdma(k).wait_send()
            x_rdma(k).wait_recv()

    return pl.pallas_call(
        body,
        out_shape=jax.ShapeDtypeStruct((m, n), jnp.bfloat16),
        in_specs=[pl.BlockSpec(memory_space=pltpu.VMEM)],
        out_specs=pl.BlockSpec(memory_space=pltpu.VMEM),
        scratch_shapes=[
            pltpu.VMEM((half, n), jnp.bfloat16),
            pltpu.VMEM((half, n), jnp.bfloat16),
            pltpu.SemaphoreType.DMA((K,)),
            pltpu.SemaphoreType.DMA((K,)),
            pltpu.SemaphoreType.DMA((K,)),
            pltpu.SemaphoreType.DMA((K,)),
        ],
        compiler_params=pltpu.CompilerParams(collective_id=0),
    )(x)


# device time: 23747 ns/iter; 1.3192x vs baseline; 1.0011x over previous
import jax
import jax.numpy as jnp
from jax import lax
from jax.experimental import pallas as pl
from jax.experimental.pallas import tpu as pltpu

K = 16


def kernel(x):
    _, m, n2 = x.shape
    n = n2 // 2
    half = m // 2
    ch = half // K

    def body(
        x_hbm,
        out_ref,
        xbuf,
        sybuf,
        rybuf,
        load_sems,
        ysend,
        yrecv,
        xsend,
        xrecv,
    ):
        my_x = lax.axis_index("x")
        my_y = lax.axis_index("y")
        peer_y = (my_x, 1 - my_y)
        peer_x = (1 - my_x, my_y)
        row0 = my_x * half

        barrier = pltpu.get_barrier_semaphore()
        for p in (peer_y, peer_x):
            pl.semaphore_signal(
                barrier, inc=1, device_id=p, device_id_type=pl.DeviceIdType.MESH
            )
        pl.semaphore_wait(barrier, 2)

        def load_dma(k):
            return pltpu.make_async_copy(
                x_hbm.at[0, pl.ds(row0 + k * ch, ch), :],
                xbuf.at[pl.ds(k * ch, ch), :],
                load_sems.at[k],
            )

        def y_rdma(k):
            return pltpu.make_async_remote_copy(
                src_ref=sybuf.at[pl.ds(k * ch, ch), :],
                dst_ref=rybuf.at[pl.ds(k * ch, ch), :],
                send_sem=ysend.at[k],
                recv_sem=yrecv.at[k],
                device_id=peer_y,
                device_id_type=pl.DeviceIdType.MESH,
            )

        def x_rdma(k):
            return pltpu.make_async_remote_copy(
                src_ref=out_ref.at[pl.ds(row0 + k * ch, ch), :],
                dst_ref=out_ref.at[pl.ds(row0 + k * ch, ch), :],
                send_sem=xsend.at[k],
                recv_sem=xrecv.at[k],
                device_id=peer_x,
                device_id_type=pl.DeviceIdType.MESH,
            )

        for k in range(K):
            load_dma(k).start()

        def stage_and_send(k):
            load_dma(k).wait()

            @pl.when(my_y == 0)
            def _():
                sybuf[pl.ds(k * ch, ch), :] = xbuf[pl.ds(k * ch, ch), n:].astype(
                    jnp.bfloat16
                )

            @pl.when(my_y == 1)
            def _():
                sybuf[pl.ds(k * ch, ch), :] = xbuf[pl.ds(k * ch, ch), :n].astype(
                    jnp.bfloat16
                )

            y_rdma(k).start()

        for k in range(K):
            stage_and_send(k)

        def reduce_and_forward(k):
            rs = row0 + k * ch
            y_rdma(k).wait_recv()

            @pl.when(my_y == 0)
            def _():
                out_ref[pl.ds(rs, ch), :] = (
                    xbuf[pl.ds(k * ch, ch), :n].astype(jnp.bfloat16)
                    + rybuf[pl.ds(k * ch, ch), :]
                )

            @pl.when(my_y == 1)
            def _():
                out_ref[pl.ds(rs, ch), :] = (
                    xbuf[pl.ds(k * ch, ch), n:].astype(jnp.bfloat16)
                    + rybuf[pl.ds(k * ch, ch), :]
                )

            x_rdma(k).start()

        for k in range(K):
            reduce_and_forward(k)

        for k in range(K):
            y_rdma(k).wait_send()
            x_rdma(k).wait_send()
            x_rdma(k).wait_recv()

    return pl.pallas_call(
        body,
        out_shape=jax.ShapeDtypeStruct((m, n), jnp.bfloat16),
        in_specs=[pl.BlockSpec(memory_space=pl.ANY)],
        out_specs=pl.BlockSpec(memory_space=pltpu.VMEM),
        scratch_shapes=[
            pltpu.VMEM((half, n2), jnp.float32),
            pltpu.VMEM((half, n), jnp.bfloat16),
            pltpu.VMEM((half, n), jnp.bfloat16),
            pltpu.SemaphoreType.DMA((K,)),
            pltpu.SemaphoreType.DMA((K,)),
            pltpu.SemaphoreType.DMA((K,)),
            pltpu.SemaphoreType.DMA((K,)),
            pltpu.SemaphoreType.DMA((K,)),
        ],
        compiler_params=pltpu.CompilerParams(collective_id=0),
    )(x)
